# baseline (speedup 1.0000x reference)
import functools

import jax
import jax.numpy as jnp
from jax import lax
from jax.experimental import pallas as pl
from jax.experimental.pallas import tpu as pltpu
from jax.experimental.pallas import tpu_sc as plsc

N = 10000
D = 128
H = 128
OUT = 64
E = 50000
W = 144

NODES = ["course", "field", "resource", "teacher", "school", "user",
         "comment", "reply", "exercise", "video"]

DST_TYPES = ["field", "resource", "teacher", "school", "user", "comment",
             "course"]
RELS1 = [
    (0, 0, 1, "course", 0),
    (1, 0, 1, "course", 1), (11, 1, 0, "exercise", 24), (12, 1, 0, "video", 25),
    (2, 0, 1, "course", 2), (10, 0, 1, "school", 10),
    (3, 0, 1, "course", 3), (9, 1, 0, "user", 22), (10, 1, 0, "teacher", 23),
    (4, 0, 1, "course", 4), (9, 0, 1, "school", 9), (7, 1, 0, "comment", 20),
    (8, 1, 0, "reply", 21),
    (5, 0, 1, "course", 5), (7, 0, 1, "user", 7), (6, 1, 0, "reply", 19),
    (0, 1, 0, "field", 13), (1, 1, 0, "resource", 14),
    (2, 1, 0, "teacher", 15), (3, 1, 0, "school", 16),
    (4, 1, 0, "user", 17), (5, 1, 0, "comment", 18),
]
NREL1 = len(RELS1)
GROUP_SIZES = [1, 3, 2, 3, 4, 3, 6]
GROUP_FIRST_K = [0, 1, 4, 6, 9, 13, 16]
GROUP_LAST_K = [0, 3, 5, 8, 12, 15, 21]

RELS2 = [(0, 1, 0, 0, 13), (1, 1, 0, 1, 14), (2, 1, 0, 2, 15),
         (3, 1, 0, 3, 16), (4, 1, 0, 4, 17), (5, 1, 0, 5, 18)]
NREL2 = len(RELS2)

NSUB = 16
NCORE = 2
CHE = 112
CHUNKS = 28
EPAD = NSUB * CHUNKS * CHE
NACC = 10016
RPT = NACC // NSUB
PADROW = N


def _sc_body(nrel, xt, srci, dsti, zrows, agg, srci_v, dsti_v, g0, g1, acc,
             gsem0, gsem1):
    c = lax.axis_index("c")
    s = lax.axis_index("s")

    @pl.loop(0, nrel // NCORE)
    def _(i):
        r = i * NCORE + c

        pltpu.sync_copy(srci.at[r, s], srci_v)
        pltpu.sync_copy(dsti.at[r, s], dsti_v)
        pltpu.async_copy(xt.at[srci_v.at[0]], g0, gsem0)
        pltpu.async_copy(xt.at[srci_v.at[1]], g1, gsem1)
        pltpu.sync_copy(zrows, acc.at[pl.ds(s * RPT, RPT)])
        plsc.subcore_barrier()

        @pl.loop(0, CHUNKS, step=2)
        def _(j):
            pltpu.make_async_copy(xt.at[srci_v.at[j]], g0, gsem0).wait()
            pltpu.sync_copy(g0, acc.at[dsti_v.at[j]], add=True)
            pltpu.async_copy(xt.at[srci_v.at[j + 2]], g0, gsem0)
            pltpu.make_async_copy(xt.at[srci_v.at[j + 1]], g1, gsem1).wait()
            pltpu.sync_copy(g1, acc.at[dsti_v.at[j + 1]], add=True)
            pltpu.async_copy(xt.at[srci_v.at[j + 3]], g1, gsem1)

        pltpu.make_async_copy(xt.at[srci_v.at[CHUNKS]], g0, gsem0).wait()
        pltpu.make_async_copy(xt.at[srci_v.at[CHUNKS + 1]], g1, gsem1).wait()

        plsc.subcore_barrier()
        base = s * RPT
        pltpu.sync_copy(acc.at[pl.ds(base, RPT)], agg.at[r, pl.ds(base, RPT)])


def _make_sc_aggregate(nrel):
    mesh = plsc.VectorSubcoreMesh(core_axis_name="c", subcore_axis_name="s")
    return pl.kernel(
        functools.partial(_sc_body, nrel),
        out_type=jax.ShapeDtypeStruct((nrel, NACC, W), jnp.float32),
        mesh=mesh,
        scratch_types=[
            pltpu.VMEM((CHUNKS + 2, CHE), jnp.int32),
            pltpu.VMEM((CHUNKS, CHE), jnp.int32),
            pltpu.VMEM((CHE, W), jnp.float32),
            pltpu.VMEM((CHE, W), jnp.float32),
            pltpu.VMEM_SHARED((NACC, W), jnp.float32),
            pltpu.SemaphoreType.DMA,
            pltpu.SemaphoreType.DMA,
        ],
        compiler_params=pltpu.CompilerParams(use_tc_tiling_on_sc=False),
    )


def _pack_edges(src_rows, dst_rows):
    nrel = src_rows.shape[0]
    src_p = jnp.zeros((nrel, EPAD), jnp.int32).at[:, :E].set(src_rows)
    dst_p = jnp.full((nrel, EPAD), PADROW, jnp.int32).at[:, :E].set(dst_rows)
    src_c = src_p.reshape(nrel, NSUB, CHUNKS, CHE)
    src_c = jnp.pad(src_c, ((0, 0), (0, 0), (0, 2), (0, 0)))
    return src_c, dst_p.reshape(nrel, NSUB, CHUNKS, CHE)


def _augment(x):
    rows = x.shape[0]
    tail = jnp.zeros((rows, W - D), x.dtype).at[:, 0].set(1.0)
    return jnp.concatenate([x, tail], axis=1)


BR = 2504
RB = NACC // BR


def _d_of_k(k):
    d = jnp.int32(0)
    for f in GROUP_FIRST_K[1:]:
        d = d + (k >= f).astype(jnp.int32)
    return d


def _is_in(k, ks):
    r = k == ks[0]
    for v in ks[1:]:
        r = jnp.logical_or(r, k == v)
    return r


def _hid_tail(n):
    lane = lax.broadcasted_iota(jnp.int32, (n, W - D), 1)
    return jnp.where(lane == 0, 1.0, 0.0).astype(jnp.float32)


def _tc1_body(agg_ref, x_ref, wl_ref, wr_ref, b_ref, out_ref):
    k = pl.program_id(1)
    is_first = _is_in(k, GROUP_FIRST_K)
    is_last = _is_in(k, GROUP_LAST_K)
    a = agg_ref[0]
    inv = 1.0 / jnp.maximum(a[:, D:D + 1], 1.0)
    contrib = jnp.dot(a[:, :D] * inv, wl_ref[0],
                      preferred_element_type=jnp.float32)

    @pl.when(is_first)
    def _():
        out_ref[0, :, :D] = (jnp.dot(x_ref[0], wr_ref[0],
                                     preferred_element_type=jnp.float32)
                             + b_ref[0] + contrib)

    @pl.when(jnp.logical_not(is_first))
    def _():
        out_ref[0, :, :D] += contrib

    @pl.when(is_last)
    def _():
        out_ref[0, :, :D] = jnp.maximum(out_ref[0, :, :D], 0.0)
        out_ref[0, :, D:] = _hid_tail(BR)


def _tc_layer1(agg, x7, wl, wr, b):
    d_of_k = _d_of_k
    grid = (RB, NREL1)
    return pl.pallas_call(
        _tc1_body,
        grid=grid,
        in_specs=[
            pl.BlockSpec((1, BR, W), lambda rb, k: (k, rb, 0)),
            pl.BlockSpec((1, BR, D), lambda rb, k: (d_of_k(k), rb, 0)),
            pl.BlockSpec((1, D, H), lambda rb, k: (k, 0, 0)),
            pl.BlockSpec((1, D, H), lambda rb, k: (d_of_k(k), 0, 0)),
            pl.BlockSpec((1, 1, H), lambda rb, k: (d_of_k(k), 0, 0)),
        ],
        out_specs=pl.BlockSpec((1, BR, W), lambda rb, k: (d_of_k(k), rb, 0)),
        out_shape=jax.ShapeDtypeStruct((len(DST_TYPES), NACC, W),
                                       jnp.float32),
    )(agg, x7, wl, wr, b)


def _tc2_body(agg_ref, h_ref, wl_ref, wr_ref, b_ref, lw_ref, lb_ref,
              out_ref, acc_ref):
    k = pl.program_id(1)
    a = agg_ref[0]
    inv = 1.0 / jnp.maximum(a[:, D:D + 1], 1.0)
    contrib = jnp.dot(a[:, :D] * inv, wl_ref[0],
                      preferred_element_type=jnp.float32)

    @pl.when(k == 0)
    def _():
        acc_ref[...] = (jnp.dot(h_ref[0, :, :D], wr_ref[...],
                                preferred_element_type=jnp.float32)
                        + b_ref[...][None, :] + contrib)

    @pl.when(k > 0)
    def _():
        acc_ref[...] += contrib

    @pl.when(k == NREL2 - 1)
    def _():
        out_ref[...] = (jnp.dot(jnp.maximum(acc_ref[...], 0.0), lw_ref[...],
                                preferred_element_type=jnp.float32)
                        + lb_ref[...][None, :])


def _tc_layer2(agg2, h, wl, wr, b, lin_w, lin_b):
    grid = (RB, NREL2)
    return pl.pallas_call(
        _tc2_body,
        grid=grid,
        in_specs=[
            pl.BlockSpec((1, BR, W), lambda rb, k: (k, rb, 0)),
            pl.BlockSpec((1, BR, W), lambda rb, k: (len(DST_TYPES) - 1, rb, 0)),
            pl.BlockSpec((1, H, H), lambda rb, k: (k, 0, 0)),
            pl.BlockSpec((H, H), lambda rb, k: (0, 0)),
            pl.BlockSpec((H,), lambda rb, k: (0,)),
            pl.BlockSpec((H, OUT), lambda rb, k: (0, 0)),
            pl.BlockSpec((OUT,), lambda rb, k: (0,)),
        ],
        out_specs=pl.BlockSpec((BR, OUT), lambda rb, k: (rb, 0)),
        out_shape=jax.ShapeDtypeStruct((NACC, OUT), jnp.float32),
        scratch_shapes=[pltpu.VMEM((BR, H), jnp.float32)],
    )(agg2, h, wl, wr, b, lin_w, lin_b)


def kernel(x_course, x_field, x_resource, x_teacher, x_school, x_user,
           x_comment, x_reply, x_exercise, x_video,
           ei_course_field, ei_course_resource, ei_course_teacher,
           ei_course_school, ei_course_user, ei_course_comment,
           ei_comment_reply, ei_user_comment, ei_user_reply,
           ei_school_user, ei_school_teacher, ei_resource_exercise,
           ei_resource_video, Wl, Wr, bl, lin_W, lin_b):
    xs = {"course": x_course, "field": x_field, "resource": x_resource,
          "teacher": x_teacher, "school": x_school, "user": x_user,
          "comment": x_comment, "reply": x_reply, "exercise": x_exercise,
          "video": x_video}
    eis = [ei_course_field, ei_course_resource, ei_course_teacher,
           ei_course_school, ei_course_user, ei_course_comment,
           ei_comment_reply, ei_user_comment, ei_user_reply,
           ei_school_user, ei_school_teacher, ei_resource_exercise,
           ei_resource_video]
    eis = [e.astype(jnp.int32) for e in eis]

    xt1 = _augment(jnp.concatenate([xs[nt] for nt in NODES], axis=0))
    src1 = jnp.stack([eis[j][sr] + N * NODES.index(snt)
                      for (j, sr, dr, snt, k) in RELS1])
    dst1 = jnp.stack([eis[j][dr] for (j, sr, dr, snt, k) in RELS1])
    srci1, dsti1 = _pack_edges(src1, dst1)
    zrows = jnp.zeros((RPT, W), jnp.float32)
    agg1 = _make_sc_aggregate(NREL1)(xt1, srci1, dsti1, zrows)

    perm1 = [k for (_, _, _, _, k) in RELS1]
    wl1 = Wl[0, jnp.asarray(perm1)]
    goff = 0
    wr_sums, b_sums = [], []
    for g in GROUP_SIZES:
        ks = jnp.asarray(perm1[goff:goff + g])
        wr_sums.append(Wr[0, ks].sum(axis=0))
        b_sums.append(bl[0, ks].sum(axis=0))
        goff += g
    wr1 = jnp.stack(wr_sums)
    b1 = jnp.stack(b_sums)[:, None, :]
    pad = ((0, NACC - N), (0, 0))
    x7 = jnp.stack([jnp.pad(xs[nt], pad) for nt in DST_TYPES])
    h = _tc_layer1(agg1, x7, wl1, wr1, b1)

    xt2 = h.reshape(len(DST_TYPES) * NACC, W)
    src2 = jnp.stack([eis[j][sr] + NACC * hi
                      for (j, sr, dr, hi, k) in RELS2])
    dst2 = jnp.stack([eis[j][dr] for (j, sr, dr, hi, k) in RELS2])
    srci2, dsti2 = _pack_edges(src2, dst2)
    agg2 = _make_sc_aggregate(NREL2)(xt2, srci2, dsti2, zrows)

    perm2 = jnp.asarray([k for (_, _, _, _, k) in RELS2])
    wl2 = Wl[1, perm2]
    wr2 = Wr[1, perm2].sum(axis=0)
    b2 = bl[1, perm2].sum(axis=0)
    y = _tc_layer2(agg2, h, wl2, wr2, b2, lin_W, lin_b)
    return y[:N]


# speedup vs baseline: 1.0488x; 1.0488x over previous
import functools

import jax
import jax.numpy as jnp
from jax import lax
from jax.experimental import pallas as pl
from jax.experimental.pallas import tpu as pltpu
from jax.experimental.pallas import tpu_sc as plsc

N = 10000
D = 128
H = 128
OUT = 64
E = 50000
W = 144

NODES = ["course", "field", "resource", "teacher", "school", "user",
         "comment", "reply", "exercise", "video"]

DST_TYPES = ["field", "resource", "teacher", "school", "user", "comment",
             "course"]
RELS1 = [
    (0, 0, 1, "course", 0),
    (1, 0, 1, "course", 1), (11, 1, 0, "exercise", 24), (12, 1, 0, "video", 25),
    (2, 0, 1, "course", 2), (10, 0, 1, "school", 10),
    (3, 0, 1, "course", 3), (9, 1, 0, "user", 22), (10, 1, 0, "teacher", 23),
    (4, 0, 1, "course", 4), (9, 0, 1, "school", 9), (7, 1, 0, "comment", 20),
    (8, 1, 0, "reply", 21),
    (5, 0, 1, "course", 5), (7, 0, 1, "user", 7), (6, 1, 0, "reply", 19),
    (0, 1, 0, "field", 13), (1, 1, 0, "resource", 14),
    (2, 1, 0, "teacher", 15), (3, 1, 0, "school", 16),
    (4, 1, 0, "user", 17), (5, 1, 0, "comment", 18),
]
NREL1 = len(RELS1)
GROUP_SIZES = [1, 3, 2, 3, 4, 3, 6]
GROUP_FIRST_K = [0, 1, 4, 6, 9, 13, 16]
GROUP_LAST_K = [0, 3, 5, 8, 12, 15, 21]

RELS2 = [(0, 1, 0, 0, 13), (1, 1, 0, 1, 14), (2, 1, 0, 2, 15),
         (3, 1, 0, 3, 16), (4, 1, 0, 4, 17), (5, 1, 0, 5, 18)]
NREL2 = len(RELS2)

NSUB = 16
NCORE = 2
CHE = 112
CHUNKS = 28
EPAD = NSUB * CHUNKS * CHE
NACC = 10016
RPT = NACC // NSUB
PADROW = N


def _sc_body(nrel, xt, srci, dsti, zrows, agg, srci_v, dsti_v, g0, g1, acc,
             gsem0, gsem1):
    c = lax.axis_index("c")
    s = lax.axis_index("s")

    @pl.loop(0, nrel // NCORE)
    def _(i):
        r = i * NCORE + c

        pltpu.sync_copy(srci.at[r, s], srci_v)
        pltpu.sync_copy(dsti.at[r, s], dsti_v)
        pltpu.async_copy(xt.at[srci_v.at[0]], g0, gsem0)
        pltpu.async_copy(xt.at[srci_v.at[1]], g1, gsem1)
        pltpu.sync_copy(zrows.at[pl.ds(s * RPT, RPT)],
                        acc.at[pl.ds(s * RPT, RPT)])
        plsc.subcore_barrier()

        @pl.loop(0, CHUNKS, step=2)
        def _(j):
            pltpu.make_async_copy(xt.at[srci_v.at[j]], g0, gsem0).wait()
            pltpu.sync_copy(g0, acc.at[dsti_v.at[j]], add=True)
            pltpu.async_copy(xt.at[srci_v.at[j + 2]], g0, gsem0)
            pltpu.make_async_copy(xt.at[srci_v.at[j + 1]], g1, gsem1).wait()
            pltpu.sync_copy(g1, acc.at[dsti_v.at[j + 1]], add=True)
            pltpu.async_copy(xt.at[srci_v.at[j + 3]], g1, gsem1)

        pltpu.make_async_copy(xt.at[srci_v.at[CHUNKS]], g0, gsem0).wait()
        pltpu.make_async_copy(xt.at[srci_v.at[CHUNKS + 1]], g1, gsem1).wait()

        plsc.subcore_barrier()
        base = s * RPT
        pltpu.sync_copy(acc.at[pl.ds(base, RPT)], agg.at[r, pl.ds(base, RPT)])


def _make_sc_aggregate(nrel):
    mesh = plsc.VectorSubcoreMesh(core_axis_name="c", subcore_axis_name="s")
    return pl.kernel(
        functools.partial(_sc_body, nrel),
        out_type=jax.ShapeDtypeStruct((nrel, NACC, W), jnp.float32),
        mesh=mesh,
        scratch_types=[
            pltpu.VMEM((CHUNKS + 2, CHE), jnp.int32),
            pltpu.VMEM((CHUNKS, CHE), jnp.int32),
            pltpu.VMEM((CHE, W), jnp.float32),
            pltpu.VMEM((CHE, W), jnp.float32),
            pltpu.VMEM_SHARED((NACC, W), jnp.float32),
            pltpu.SemaphoreType.DMA,
            pltpu.SemaphoreType.DMA,
        ],
        compiler_params=pltpu.CompilerParams(use_tc_tiling_on_sc=False),
    )


def _pack_edges(src_rows, dst_rows):
    nrel = src_rows.shape[0]
    src_p = jnp.zeros((nrel, EPAD), jnp.int32).at[:, :E].set(src_rows)
    dst_p = jnp.full((nrel, EPAD), PADROW, jnp.int32).at[:, :E].set(dst_rows)
    src_c = src_p.reshape(nrel, NSUB, CHUNKS, CHE)
    src_c = jnp.pad(src_c, ((0, 0), (0, 0), (0, 2), (0, 0)))
    return src_c, dst_p.reshape(nrel, NSUB, CHUNKS, CHE)


def _augment(x):
    rows = x.shape[0]
    tail = jnp.zeros((rows, W - D), x.dtype).at[:, 0].set(1.0)
    return jnp.concatenate([x, tail], axis=1)


BR = 2504
RB = NACC // BR


def _d_of_k(k):
    d = jnp.int32(0)
    for f in GROUP_FIRST_K[1:]:
        d = d + (k >= f).astype(jnp.int32)
    return d


def _is_in(k, ks):
    r = k == ks[0]
    for v in ks[1:]:
        r = jnp.logical_or(r, k == v)
    return r


def _hid_tail(n):
    lane = lax.broadcasted_iota(jnp.int32, (n, W - D), 1)
    return jnp.where(lane == 0, 1.0, 0.0).astype(jnp.float32)


def _tc1_body(agg_ref, x_ref, wl_ref, wr_ref, b_ref, out_ref):
    k = pl.program_id(1)
    is_first = _is_in(k, GROUP_FIRST_K)
    is_last = _is_in(k, GROUP_LAST_K)
    a = agg_ref[0]
    inv = 1.0 / jnp.maximum(a[:, D:D + 1], 1.0)
    contrib = jnp.dot(a[:, :D] * inv, wl_ref[0],
                      preferred_element_type=jnp.float32)

    @pl.when(is_first)
    def _():
        out_ref[0, :, :D] = (jnp.dot(x_ref[0], wr_ref[0],
                                     preferred_element_type=jnp.float32)
                             + b_ref[0] + contrib)

    @pl.when(jnp.logical_not(is_first))
    def _():
        out_ref[0, :, :D] += contrib

    @pl.when(is_last)
    def _():
        out_ref[0, :, :D] = jnp.maximum(out_ref[0, :, :D], 0.0)
        out_ref[0, :, D:] = _hid_tail(BR)


def _tc_layer1(agg, x7, wl, wr, b):
    d_of_k = _d_of_k
    grid = (RB, NREL1)
    return pl.pallas_call(
        _tc1_body,
        grid=grid,
        in_specs=[
            pl.BlockSpec((1, BR, W), lambda rb, k: (k, rb, 0)),
            pl.BlockSpec((1, BR, D), lambda rb, k: (d_of_k(k), rb, 0)),
            pl.BlockSpec((1, D, H), lambda rb, k: (k, 0, 0)),
            pl.BlockSpec((1, D, H), lambda rb, k: (d_of_k(k), 0, 0)),
            pl.BlockSpec((1, 1, H), lambda rb, k: (d_of_k(k), 0, 0)),
        ],
        out_specs=pl.BlockSpec((1, BR, W), lambda rb, k: (d_of_k(k), rb, 0)),
        out_shape=jax.ShapeDtypeStruct((len(DST_TYPES), NACC, W),
                                       jnp.float32),
    )(agg, x7, wl, wr, b)


def _tc2_body(agg_ref, h_ref, wl_ref, wr_ref, b_ref, lw_ref, lb_ref,
              out_ref, acc_ref):
    k = pl.program_id(1)
    a = agg_ref[0]
    inv = 1.0 / jnp.maximum(a[:, D:D + 1], 1.0)
    contrib = jnp.dot(a[:, :D] * inv, wl_ref[0],
                      preferred_element_type=jnp.float32)

    @pl.when(k == 0)
    def _():
        acc_ref[...] = (jnp.dot(h_ref[0, :, :D], wr_ref[...],
                                preferred_element_type=jnp.float32)
                        + b_ref[...][None, :] + contrib)

    @pl.when(k > 0)
    def _():
        acc_ref[...] += contrib

    @pl.when(k == NREL2 - 1)
    def _():
        out_ref[...] = (jnp.dot(jnp.maximum(acc_ref[...], 0.0), lw_ref[...],
                                preferred_element_type=jnp.float32)
                        + lb_ref[...][None, :])


def _tc_layer2(agg2, h, wl, wr, b, lin_w, lin_b):
    grid = (RB, NREL2)
    return pl.pallas_call(
        _tc2_body,
        grid=grid,
        in_specs=[
            pl.BlockSpec((1, BR, W), lambda rb, k: (k, rb, 0)),
            pl.BlockSpec((1, BR, W), lambda rb, k: (len(DST_TYPES) - 1, rb, 0)),
            pl.BlockSpec((1, H, H), lambda rb, k: (k, 0, 0)),
            pl.BlockSpec((H, H), lambda rb, k: (0, 0)),
            pl.BlockSpec((H,), lambda rb, k: (0,)),
            pl.BlockSpec((H, OUT), lambda rb, k: (0, 0)),
            pl.BlockSpec((OUT,), lambda rb, k: (0,)),
        ],
        out_specs=pl.BlockSpec((BR, OUT), lambda rb, k: (rb, 0)),
        out_shape=jax.ShapeDtypeStruct((NACC, OUT), jnp.float32),
        scratch_shapes=[pltpu.VMEM((BR, H), jnp.float32)],
    )(agg2, h, wl, wr, b, lin_w, lin_b)


def kernel(x_course, x_field, x_resource, x_teacher, x_school, x_user,
           x_comment, x_reply, x_exercise, x_video,
           ei_course_field, ei_course_resource, ei_course_teacher,
           ei_course_school, ei_course_user, ei_course_comment,
           ei_comment_reply, ei_user_comment, ei_user_reply,
           ei_school_user, ei_school_teacher, ei_resource_exercise,
           ei_resource_video, Wl, Wr, bl, lin_W, lin_b):
    xs = {"course": x_course, "field": x_field, "resource": x_resource,
          "teacher": x_teacher, "school": x_school, "user": x_user,
          "comment": x_comment, "reply": x_reply, "exercise": x_exercise,
          "video": x_video}
    eis = [ei_course_field, ei_course_resource, ei_course_teacher,
           ei_course_school, ei_course_user, ei_course_comment,
           ei_comment_reply, ei_user_comment, ei_user_reply,
           ei_school_user, ei_school_teacher, ei_resource_exercise,
           ei_resource_video]
    eis = [e.astype(jnp.int32) for e in eis]

    xt1 = _augment(jnp.concatenate([xs[nt] for nt in NODES], axis=0))
    src1 = jnp.stack([eis[j][sr] + N * NODES.index(snt)
                      for (j, sr, dr, snt, k) in RELS1])
    dst1 = jnp.stack([eis[j][dr] for (j, sr, dr, snt, k) in RELS1])
    srci1, dsti1 = _pack_edges(src1, dst1)
    zrows = jnp.zeros((NACC, W), jnp.float32)
    agg1 = _make_sc_aggregate(NREL1)(xt1, srci1, dsti1, zrows)

    perm1 = [k for (_, _, _, _, k) in RELS1]
    wl1 = Wl[0, jnp.asarray(perm1)]
    goff = 0
    wr_sums, b_sums = [], []
    for g in GROUP_SIZES:
        ks = jnp.asarray(perm1[goff:goff + g])
        wr_sums.append(Wr[0, ks].sum(axis=0))
        b_sums.append(bl[0, ks].sum(axis=0))
        goff += g
    wr1 = jnp.stack(wr_sums)
    b1 = jnp.stack(b_sums)[:, None, :]
    pad = ((0, NACC - N), (0, 0))
    x7 = jnp.stack([jnp.pad(xs[nt], pad) for nt in DST_TYPES])
    h = _tc_layer1(agg1, x7, wl1, wr1, b1)

    xt2 = h.reshape(len(DST_TYPES) * NACC, W)
    src2 = jnp.stack([eis[j][sr] + NACC * hi
                      for (j, sr, dr, hi, k) in RELS2])
    dst2 = jnp.stack([eis[j][dr] for (j, sr, dr, hi, k) in RELS2])
    srci2, dsti2 = _pack_edges(src2, dst2)
    agg2 = _make_sc_aggregate(NREL2)(xt2, srci2, dsti2, zrows)

    perm2 = jnp.asarray([k for (_, _, _, _, k) in RELS2])
    wl2 = Wl[1, perm2]
    wr2 = Wr[1, perm2].sum(axis=0)
    b2 = bl[1, perm2].sum(axis=0)
    y = _tc_layer2(agg2, h, wl2, wr2, b2, lin_W, lin_b)
    return y[:N]


# speedup vs baseline: 2.0576x; 1.9618x over previous
import functools

import jax
import jax.numpy as jnp
from jax import lax
from jax.experimental import pallas as pl
from jax.experimental.pallas import tpu as pltpu
from jax.experimental.pallas import tpu_sc as plsc

N = 10000
D = 128
H = 128
OUT = 64
E = 50000
W = 144

NODES = ["course", "field", "resource", "teacher", "school", "user",
         "comment", "reply", "exercise", "video"]

DST_TYPES = ["field", "resource", "teacher", "school", "user", "comment",
             "course"]
RELS1 = [
    (0, 0, 1, "course", 0),
    (1, 0, 1, "course", 1), (11, 1, 0, "exercise", 24), (12, 1, 0, "video", 25),
    (2, 0, 1, "course", 2), (10, 0, 1, "school", 10),
    (3, 0, 1, "course", 3), (9, 1, 0, "user", 22), (10, 1, 0, "teacher", 23),
    (4, 0, 1, "course", 4), (9, 0, 1, "school", 9), (7, 1, 0, "comment", 20),
    (8, 1, 0, "reply", 21),
    (5, 0, 1, "course", 5), (7, 0, 1, "user", 7), (6, 1, 0, "reply", 19),
    (0, 1, 0, "field", 13), (1, 1, 0, "resource", 14),
    (2, 1, 0, "teacher", 15), (3, 1, 0, "school", 16),
    (4, 1, 0, "user", 17), (5, 1, 0, "comment", 18),
]
NREL1 = len(RELS1)
GROUP_SIZES = [1, 3, 2, 3, 4, 3, 6]
GROUP_FIRST_K = [0, 1, 4, 6, 9, 13, 16]
GROUP_LAST_K = [0, 3, 5, 8, 12, 15, 21]

RELS2 = [(0, 1, 0, 0, 13), (1, 1, 0, 1, 14), (2, 1, 0, 2, 15),
         (3, 1, 0, 3, 16), (4, 1, 0, 4, 17), (5, 1, 0, 5, 18)]
NREL2 = len(RELS2)

NSUB = 16
NCORE = 2
CHE = 128
CHUNKS = 25
EPAD = NSUB * CHUNKS * CHE
NACC = 10016
RPT = NACC // NSUB
PADROW = N


def _sc_body(nrel, xt, srci, dsti, zrows, agg, srcj0, srcj1, dstj, g0, g1,
             acc, gsem0, gsem1):
    c = lax.axis_index("c")
    s = lax.axis_index("s")
    srcjs = (srcj0, srcj1)
    gs = (g0, g1)
    sems = (gsem0, gsem1)

    @pl.loop(0, nrel // NCORE)
    def _(i):
        r = i * NCORE + c

        pltpu.sync_copy(zrows.at[pl.ds(s * RPT, RPT)],
                        acc.at[pl.ds(s * RPT, RPT)])
        pltpu.sync_copy(srci.at[r, s, 0], srcj0)
        descs = {0: pltpu.async_copy(xt.at[srcj0], g0, gsem0)}
        plsc.subcore_barrier()

        for j in range(CHUNKS):
            b = j % 2
            if j + 1 < CHUNKS:
                pltpu.sync_copy(srci.at[r, s, j + 1], srcjs[1 - b])
                descs[j + 1] = pltpu.async_copy(xt.at[srcjs[1 - b]],
                                                gs[1 - b], sems[1 - b])
            pltpu.sync_copy(dsti.at[r, s, j], dstj)
            descs.pop(j).wait()
            pltpu.sync_copy(gs[b], acc.at[dstj], add=True)

        plsc.subcore_barrier()
        base = s * RPT
        pltpu.sync_copy(acc.at[pl.ds(base, RPT)], agg.at[r, pl.ds(base, RPT)])


def _make_sc_aggregate(nrel):
    mesh = plsc.VectorSubcoreMesh(core_axis_name="c", subcore_axis_name="s")
    return pl.kernel(
        functools.partial(_sc_body, nrel),
        out_type=jax.ShapeDtypeStruct((nrel, NACC, W), jnp.float32),
        mesh=mesh,
        scratch_types=[
            pltpu.VMEM((CHE,), jnp.int32),
            pltpu.VMEM((CHE,), jnp.int32),
            pltpu.VMEM((CHE,), jnp.int32),
            pltpu.VMEM((CHE, W), jnp.float32),
            pltpu.VMEM((CHE, W), jnp.float32),
            pltpu.VMEM_SHARED((NACC, W), jnp.float32),
            pltpu.SemaphoreType.DMA,
            pltpu.SemaphoreType.DMA,
        ],
        compiler_params=pltpu.CompilerParams(use_tc_tiling_on_sc=False),
    )


def _pack_edges(src_rows, dst_rows):
    nrel = src_rows.shape[0]
    src_p = jnp.zeros((nrel, EPAD), jnp.int32).at[:, :E].set(src_rows)
    dst_p = jnp.full((nrel, EPAD), PADROW, jnp.int32).at[:, :E].set(dst_rows)
    return (src_p.reshape(nrel, NSUB, CHUNKS, CHE),
            dst_p.reshape(nrel, NSUB, CHUNKS, CHE))


def _augment(x):
    rows = x.shape[0]
    tail = jnp.zeros((rows, W - D), x.dtype).at[:, 0].set(1.0)
    return jnp.concatenate([x, tail], axis=1)


BR = 2504
RB = NACC // BR


def _d_of_k(k):
    d = jnp.int32(0)
    for f in GROUP_FIRST_K[1:]:
        d = d + (k >= f).astype(jnp.int32)
    return d


def _is_in(k, ks):
    r = k == ks[0]
    for v in ks[1:]:
        r = jnp.logical_or(r, k == v)
    return r


def _hid_tail(n):
    lane = lax.broadcasted_iota(jnp.int32, (n, W - D), 1)
    return jnp.where(lane == 0, 1.0, 0.0).astype(jnp.float32)


def _tc1_body(agg_ref, x_ref, wl_ref, wr_ref, b_ref, out_ref):
    k = pl.program_id(1)
    is_first = _is_in(k, GROUP_FIRST_K)
    is_last = _is_in(k, GROUP_LAST_K)
    a = agg_ref[0]
    inv = 1.0 / jnp.maximum(a[:, D:D + 1], 1.0)
    contrib = jnp.dot(a[:, :D] * inv, wl_ref[0],
                      preferred_element_type=jnp.float32)

    @pl.when(is_first)
    def _():
        out_ref[0, :, :D] = (jnp.dot(x_ref[0], wr_ref[0],
                                     preferred_element_type=jnp.float32)
                             + b_ref[0] + contrib)

    @pl.when(jnp.logical_not(is_first))
    def _():
        out_ref[0, :, :D] += contrib

    @pl.when(is_last)
    def _():
        out_ref[0, :, :D] = jnp.maximum(out_ref[0, :, :D], 0.0)
        out_ref[0, :, D:] = _hid_tail(BR)


def _tc_layer1(agg, x7, wl, wr, b):
    d_of_k = _d_of_k
    grid = (RB, NREL1)
    return pl.pallas_call(
        _tc1_body,
        grid=grid,
        in_specs=[
            pl.BlockSpec((1, BR, W), lambda rb, k: (k, rb, 0)),
            pl.BlockSpec((1, BR, D), lambda rb, k: (d_of_k(k), rb, 0)),
            pl.BlockSpec((1, D, H), lambda rb, k: (k, 0, 0)),
            pl.BlockSpec((1, D, H), lambda rb, k: (d_of_k(k), 0, 0)),
            pl.BlockSpec((1, 1, H), lambda rb, k: (d_of_k(k), 0, 0)),
        ],
        out_specs=pl.BlockSpec((1, BR, W), lambda rb, k: (d_of_k(k), rb, 0)),
        out_shape=jax.ShapeDtypeStruct((len(DST_TYPES), NACC, W),
                                       jnp.float32),
    )(agg, x7, wl, wr, b)


def _tc2_body(agg_ref, h_ref, wl_ref, wr_ref, b_ref, lw_ref, lb_ref,
              out_ref, acc_ref):
    k = pl.program_id(1)
    a = agg_ref[0]
    inv = 1.0 / jnp.maximum(a[:, D:D + 1], 1.0)
    contrib = jnp.dot(a[:, :D] * inv, wl_ref[0],
                      preferred_element_type=jnp.float32)

    @pl.when(k == 0)
    def _():
        acc_ref[...] = (jnp.dot(h_ref[0, :, :D], wr_ref[...],
                                preferred_element_type=jnp.float32)
                        + b_ref[...][None, :] + contrib)

    @pl.when(k > 0)
    def _():
        acc_ref[...] += contrib

    @pl.when(k == NREL2 - 1)
    def _():
        out_ref[...] = (jnp.dot(jnp.maximum(acc_ref[...], 0.0), lw_ref[...],
                                preferred_element_type=jnp.float32)
                        + lb_ref[...][None, :])


def _tc_layer2(agg2, h, wl, wr, b, lin_w, lin_b):
    grid = (RB, NREL2)
    return pl.pallas_call(
        _tc2_body,
        grid=grid,
        in_specs=[
            pl.BlockSpec((1, BR, W), lambda rb, k: (k, rb, 0)),
            pl.BlockSpec((1, BR, W), lambda rb, k: (len(DST_TYPES) - 1, rb, 0)),
            pl.BlockSpec((1, H, H), lambda rb, k: (k, 0, 0)),
            pl.BlockSpec((H, H), lambda rb, k: (0, 0)),
            pl.BlockSpec((H,), lambda rb, k: (0,)),
            pl.BlockSpec((H, OUT), lambda rb, k: (0, 0)),
            pl.BlockSpec((OUT,), lambda rb, k: (0,)),
        ],
        out_specs=pl.BlockSpec((BR, OUT), lambda rb, k: (rb, 0)),
        out_shape=jax.ShapeDtypeStruct((NACC, OUT), jnp.float32),
        scratch_shapes=[pltpu.VMEM((BR, H), jnp.float32)],
    )(agg2, h, wl, wr, b, lin_w, lin_b)


def kernel(x_course, x_field, x_resource, x_teacher, x_school, x_user,
           x_comment, x_reply, x_exercise, x_video,
           ei_course_field, ei_course_resource, ei_course_teacher,
           ei_course_school, ei_course_user, ei_course_comment,
           ei_comment_reply, ei_user_comment, ei_user_reply,
           ei_school_user, ei_school_teacher, ei_resource_exercise,
           ei_resource_video, Wl, Wr, bl, lin_W, lin_b):
    xs = {"course": x_course, "field": x_field, "resource": x_resource,
          "teacher": x_teacher, "school": x_school, "user": x_user,
          "comment": x_comment, "reply": x_reply, "exercise": x_exercise,
          "video": x_video}
    eis = [ei_course_field, ei_course_resource, ei_course_teacher,
           ei_course_school, ei_course_user, ei_course_comment,
           ei_comment_reply, ei_user_comment, ei_user_reply,
           ei_school_user, ei_school_teacher, ei_resource_exercise,
           ei_resource_video]
    eis = [e.astype(jnp.int32) for e in eis]

    xt1 = _augment(jnp.concatenate([xs[nt] for nt in NODES], axis=0))
    src1 = jnp.stack([eis[j][sr] + N * NODES.index(snt)
                      for (j, sr, dr, snt, k) in RELS1])
    dst1 = jnp.stack([eis[j][dr] for (j, sr, dr, snt, k) in RELS1])
    srci1, dsti1 = _pack_edges(src1, dst1)
    zrows = jnp.zeros((NACC, W), jnp.float32)
    agg1 = _make_sc_aggregate(NREL1)(xt1, srci1, dsti1, zrows)

    perm1 = [k for (_, _, _, _, k) in RELS1]
    wl1 = Wl[0, jnp.asarray(perm1)]
    goff = 0
    wr_sums, b_sums = [], []
    for g in GROUP_SIZES:
        ks = jnp.asarray(perm1[goff:goff + g])
        wr_sums.append(Wr[0, ks].sum(axis=0))
        b_sums.append(bl[0, ks].sum(axis=0))
        goff += g
    wr1 = jnp.stack(wr_sums)
    b1 = jnp.stack(b_sums)[:, None, :]
    pad = ((0, NACC - N), (0, 0))
    x7 = jnp.stack([jnp.pad(xs[nt], pad) for nt in DST_TYPES])
    h = _tc_layer1(agg1, x7, wl1, wr1, b1)

    xt2 = h.reshape(len(DST_TYPES) * NACC, W)
    src2 = jnp.stack([eis[j][sr] + NACC * hi
                      for (j, sr, dr, hi, k) in RELS2])
    dst2 = jnp.stack([eis[j][dr] for (j, sr, dr, hi, k) in RELS2])
    srci2, dsti2 = _pack_edges(src2, dst2)
    agg2 = _make_sc_aggregate(NREL2)(xt2, srci2, dsti2, zrows)

    perm2 = jnp.asarray([k for (_, _, _, _, k) in RELS2])
    wl2 = Wl[1, perm2]
    wr2 = Wr[1, perm2].sum(axis=0)
    b2 = bl[1, perm2].sum(axis=0)
    y = _tc_layer2(agg2, h, wl2, wr2, b2, lin_W, lin_b)
    return y[:N]


# speedup vs baseline: 3.5385x; 1.7197x over previous
import functools

import jax
import jax.numpy as jnp
from jax import lax
from jax.experimental import pallas as pl
from jax.experimental.pallas import tpu as pltpu
from jax.experimental.pallas import tpu_sc as plsc

N = 10000
D = 128
H = 128
OUT = 64
E = 50000
W = 144

NODES = ["course", "field", "resource", "teacher", "school", "user",
         "comment", "reply", "exercise", "video"]

DST_TYPES = ["field", "resource", "teacher", "school", "user", "comment",
             "course"]
RELS1 = [
    (0, 0, 1, "course", 0),
    (1, 0, 1, "course", 1), (11, 1, 0, "exercise", 24), (12, 1, 0, "video", 25),
    (2, 0, 1, "course", 2), (10, 0, 1, "school", 10),
    (3, 0, 1, "course", 3), (9, 1, 0, "user", 22), (10, 1, 0, "teacher", 23),
    (4, 0, 1, "course", 4), (9, 0, 1, "school", 9), (7, 1, 0, "comment", 20),
    (8, 1, 0, "reply", 21),
    (5, 0, 1, "course", 5), (7, 0, 1, "user", 7), (6, 1, 0, "reply", 19),
    (0, 1, 0, "field", 13), (1, 1, 0, "resource", 14),
    (2, 1, 0, "teacher", 15), (3, 1, 0, "school", 16),
    (4, 1, 0, "user", 17), (5, 1, 0, "comment", 18),
]
NREL1 = len(RELS1)
GROUP_SIZES = [1, 3, 2, 3, 4, 3, 6]
GROUP_FIRST_K = [0, 1, 4, 6, 9, 13, 16]
GROUP_LAST_K = [0, 3, 5, 8, 12, 15, 21]

RELS2 = [(0, 1, 0, 0, 13), (1, 1, 0, 1, 14), (2, 1, 0, 2, 15),
         (3, 1, 0, 3, 16), (4, 1, 0, 4, 17), (5, 1, 0, 5, 18)]
NREL2 = len(RELS2)

NSUB = 16
NCORE = 2
CHE = 112
CHUNKS = 28
EPAD = NSUB * CHUNKS * CHE
NACC = 10016
RPT = NACC // NSUB
PADROW = N


def _sc_body(nrel, xt, srci, dsti, zrows, agg, srci_v, dsti_v, g0, g1,
             acc, gsem0, gsem1):
    c = lax.axis_index("c")
    s = lax.axis_index("s")
    gs = (g0, g1)
    sems = (gsem0, gsem1)

    @pl.loop(0, nrel // NCORE)
    def _(i):
        r = i * NCORE + c

        pltpu.sync_copy(zrows.at[pl.ds(s * RPT, RPT)],
                        acc.at[pl.ds(s * RPT, RPT)])
        pltpu.sync_copy(srci.at[r, s], srci_v)
        pltpu.sync_copy(dsti.at[r, s], dsti_v)
        descs = {0: pltpu.async_copy(xt.at[srci_v.at[0]], g0, gsem0)}
        plsc.subcore_barrier()

        for j in range(CHUNKS):
            b = j % 2
            if j + 1 < CHUNKS:
                descs[j + 1] = pltpu.async_copy(xt.at[srci_v.at[j + 1]],
                                                gs[1 - b], sems[1 - b])
            descs.pop(j).wait()
            pltpu.sync_copy(gs[b], acc.at[dsti_v.at[j]], add=True)

        plsc.subcore_barrier()
        base = s * RPT
        pltpu.sync_copy(acc.at[pl.ds(base, RPT)], agg.at[r, pl.ds(base, RPT)])


def _make_sc_aggregate(nrel):
    mesh = plsc.VectorSubcoreMesh(core_axis_name="c", subcore_axis_name="s")
    return pl.kernel(
        functools.partial(_sc_body, nrel),
        out_type=jax.ShapeDtypeStruct((nrel, NACC, W), jnp.float32),
        mesh=mesh,
        scratch_types=[
            pltpu.VMEM((CHUNKS, CHE), jnp.int32),
            pltpu.VMEM((CHUNKS, CHE), jnp.int32),
            pltpu.VMEM((CHE, W), jnp.float32),
            pltpu.VMEM((CHE, W), jnp.float32),
            pltpu.VMEM_SHARED((NACC, W), jnp.float32),
            pltpu.SemaphoreType.DMA,
            pltpu.SemaphoreType.DMA,
        ],
        compiler_params=pltpu.CompilerParams(use_tc_tiling_on_sc=False),
    )


def _pack_edges(src_rows, dst_rows):
    nrel = src_rows.shape[0]
    src_p = jnp.zeros((nrel, EPAD), jnp.int32).at[:, :E].set(src_rows)
    dst_p = jnp.full((nrel, EPAD), PADROW, jnp.int32).at[:, :E].set(dst_rows)
    return (src_p.reshape(nrel, NSUB, CHUNKS, CHE),
            dst_p.reshape(nrel, NSUB, CHUNKS, CHE))


def _augment(x):
    rows = x.shape[0]
    tail = jnp.zeros((rows, W - D), x.dtype).at[:, 0].set(1.0)
    return jnp.concatenate([x, tail], axis=1)


BR = 2504
RB = NACC // BR


def _d_of_k(k):
    d = jnp.int32(0)
    for f in GROUP_FIRST_K[1:]:
        d = d + (k >= f).astype(jnp.int32)
    return d


def _is_in(k, ks):
    r = k == ks[0]
    for v in ks[1:]:
        r = jnp.logical_or(r, k == v)
    return r


def _hid_tail(n):
    lane = lax.broadcasted_iota(jnp.int32, (n, W - D), 1)
    return jnp.where(lane == 0, 1.0, 0.0).astype(jnp.float32)


def _tc1_body(agg_ref, x_ref, wl_ref, wr_ref, b_ref, out_ref):
    k = pl.program_id(1)
    is_first = _is_in(k, GROUP_FIRST_K)
    is_last = _is_in(k, GROUP_LAST_K)
    a = agg_ref[0]
    inv = 1.0 / jnp.maximum(a[:, D:D + 1], 1.0)
    contrib = jnp.dot(a[:, :D] * inv, wl_ref[0],
                      preferred_element_type=jnp.float32)

    @pl.when(is_first)
    def _():
        out_ref[0, :, :D] = (jnp.dot(x_ref[0], wr_ref[0],
                                     preferred_element_type=jnp.float32)
                             + b_ref[0] + contrib)

    @pl.when(jnp.logical_not(is_first))
    def _():
        out_ref[0, :, :D] += contrib

    @pl.when(is_last)
    def _():
        out_ref[0, :, :D] = jnp.maximum(out_ref[0, :, :D], 0.0)
        out_ref[0, :, D:] = _hid_tail(BR)


def _tc_layer1(agg, x7, wl, wr, b):
    d_of_k = _d_of_k
    grid = (RB, NREL1)
    return pl.pallas_call(
        _tc1_body,
        grid=grid,
        in_specs=[
            pl.BlockSpec((1, BR, W), lambda rb, k: (k, rb, 0)),
            pl.BlockSpec((1, BR, D), lambda rb, k: (d_of_k(k), rb, 0)),
            pl.BlockSpec((1, D, H), lambda rb, k: (k, 0, 0)),
            pl.BlockSpec((1, D, H), lambda rb, k: (d_of_k(k), 0, 0)),
            pl.BlockSpec((1, 1, H), lambda rb, k: (d_of_k(k), 0, 0)),
        ],
        out_specs=pl.BlockSpec((1, BR, W), lambda rb, k: (d_of_k(k), rb, 0)),
        out_shape=jax.ShapeDtypeStruct((len(DST_TYPES), NACC, W),
                                       jnp.float32),
    )(agg, x7, wl, wr, b)


def _tc2_body(agg_ref, h_ref, wl_ref, wr_ref, b_ref, lw_ref, lb_ref,
              out_ref, acc_ref):
    k = pl.program_id(1)
    a = agg_ref[0]
    inv = 1.0 / jnp.maximum(a[:, D:D + 1], 1.0)
    contrib = jnp.dot(a[:, :D] * inv, wl_ref[0],
                      preferred_element_type=jnp.float32)

    @pl.when(k == 0)
    def _():
        acc_ref[...] = (jnp.dot(h_ref[0, :, :D], wr_ref[...],
                                preferred_element_type=jnp.float32)
                        + b_ref[...][None, :] + contrib)

    @pl.when(k > 0)
    def _():
        acc_ref[...] += contrib

    @pl.when(k == NREL2 - 1)
    def _():
        out_ref[...] = (jnp.dot(jnp.maximum(acc_ref[...], 0.0), lw_ref[...],
                                preferred_element_type=jnp.float32)
                        + lb_ref[...][None, :])


def _tc_layer2(agg2, h, wl, wr, b, lin_w, lin_b):
    grid = (RB, NREL2)
    return pl.pallas_call(
        _tc2_body,
        grid=grid,
        in_specs=[
            pl.BlockSpec((1, BR, W), lambda rb, k: (k, rb, 0)),
            pl.BlockSpec((1, BR, W), lambda rb, k: (len(DST_TYPES) - 1, rb, 0)),
            pl.BlockSpec((1, H, H), lambda rb, k: (k, 0, 0)),
            pl.BlockSpec((H, H), lambda rb, k: (0, 0)),
            pl.BlockSpec((H,), lambda rb, k: (0,)),
            pl.BlockSpec((H, OUT), lambda rb, k: (0, 0)),
            pl.BlockSpec((OUT,), lambda rb, k: (0,)),
        ],
        out_specs=pl.BlockSpec((BR, OUT), lambda rb, k: (rb, 0)),
        out_shape=jax.ShapeDtypeStruct((NACC, OUT), jnp.float32),
        scratch_shapes=[pltpu.VMEM((BR, H), jnp.float32)],
    )(agg2, h, wl, wr, b, lin_w, lin_b)


def kernel(x_course, x_field, x_resource, x_teacher, x_school, x_user,
           x_comment, x_reply, x_exercise, x_video,
           ei_course_field, ei_course_resource, ei_course_teacher,
           ei_course_school, ei_course_user, ei_course_comment,
           ei_comment_reply, ei_user_comment, ei_user_reply,
           ei_school_user, ei_school_teacher, ei_resource_exercise,
           ei_resource_video, Wl, Wr, bl, lin_W, lin_b):
    xs = {"course": x_course, "field": x_field, "resource": x_resource,
          "teacher": x_teacher, "school": x_school, "user": x_user,
          "comment": x_comment, "reply": x_reply, "exercise": x_exercise,
          "video": x_video}
    eis = [ei_course_field, ei_course_resource, ei_course_teacher,
           ei_course_school, ei_course_user, ei_course_comment,
           ei_comment_reply, ei_user_comment, ei_user_reply,
           ei_school_user, ei_school_teacher, ei_resource_exercise,
           ei_resource_video]
    eis = [e.astype(jnp.int32) for e in eis]

    xt1 = _augment(jnp.concatenate([xs[nt] for nt in NODES], axis=0))
    src1 = jnp.stack([eis[j][sr] + N * NODES.index(snt)
                      for (j, sr, dr, snt, k) in RELS1])
    dst1 = jnp.stack([eis[j][dr] for (j, sr, dr, snt, k) in RELS1])
    srci1, dsti1 = _pack_edges(src1, dst1)
    zrows = jnp.zeros((NACC, W), jnp.float32)
    agg1 = _make_sc_aggregate(NREL1)(xt1, srci1, dsti1, zrows)

    perm1 = [k for (_, _, _, _, k) in RELS1]
    wl1 = Wl[0, jnp.asarray(perm1)]
    goff = 0
    wr_sums, b_sums = [], []
    for g in GROUP_SIZES:
        ks = jnp.asarray(perm1[goff:goff + g])
        wr_sums.append(Wr[0, ks].sum(axis=0))
        b_sums.append(bl[0, ks].sum(axis=0))
        goff += g
    wr1 = jnp.stack(wr_sums)
    b1 = jnp.stack(b_sums)[:, None, :]
    pad = ((0, NACC - N), (0, 0))
    x7 = jnp.stack([jnp.pad(xs[nt], pad) for nt in DST_TYPES])
    h = _tc_layer1(agg1, x7, wl1, wr1, b1)

    xt2 = h.reshape(len(DST_TYPES) * NACC, W)
    src2 = jnp.stack([eis[j][sr] + NACC * hi
                      for (j, sr, dr, hi, k) in RELS2])
    dst2 = jnp.stack([eis[j][dr] for (j, sr, dr, hi, k) in RELS2])
    srci2, dsti2 = _pack_edges(src2, dst2)
    agg2 = _make_sc_aggregate(NREL2)(xt2, srci2, dsti2, zrows)

    perm2 = jnp.asarray([k for (_, _, _, _, k) in RELS2])
    wl2 = Wl[1, perm2]
    wr2 = Wr[1, perm2].sum(axis=0)
    b2 = bl[1, perm2].sum(axis=0)
    y = _tc_layer2(agg2, h, wl2, wr2, b2, lin_W, lin_b)
    return y[:N]


# speedup vs baseline: 3.5431x; 1.0013x over previous
import functools

import jax
import jax.numpy as jnp
from jax import lax
from jax.experimental import pallas as pl
from jax.experimental.pallas import tpu as pltpu
from jax.experimental.pallas import tpu_sc as plsc

N = 10000
D = 128
H = 128
OUT = 64
E = 50000
W = 144

NODES = ["course", "field", "resource", "teacher", "school", "user",
         "comment", "reply", "exercise", "video"]

DST_TYPES = ["field", "resource", "teacher", "school", "user", "comment",
             "course"]
RELS1 = [
    (0, 0, 1, "course", 0),
    (1, 0, 1, "course", 1), (11, 1, 0, "exercise", 24), (12, 1, 0, "video", 25),
    (2, 0, 1, "course", 2), (10, 0, 1, "school", 10),
    (3, 0, 1, "course", 3), (9, 1, 0, "user", 22), (10, 1, 0, "teacher", 23),
    (4, 0, 1, "course", 4), (9, 0, 1, "school", 9), (7, 1, 0, "comment", 20),
    (8, 1, 0, "reply", 21),
    (5, 0, 1, "course", 5), (7, 0, 1, "user", 7), (6, 1, 0, "reply", 19),
    (0, 1, 0, "field", 13), (1, 1, 0, "resource", 14),
    (2, 1, 0, "teacher", 15), (3, 1, 0, "school", 16),
    (4, 1, 0, "user", 17), (5, 1, 0, "comment", 18),
]
NREL1 = len(RELS1)
GROUP_SIZES = [1, 3, 2, 3, 4, 3, 6]
GROUP_FIRST_K = [0, 1, 4, 6, 9, 13, 16]
GROUP_LAST_K = [0, 3, 5, 8, 12, 15, 21]

RELS2 = [(0, 1, 0, 0, 13), (1, 1, 0, 1, 14), (2, 1, 0, 2, 15),
         (3, 1, 0, 3, 16), (4, 1, 0, 4, 17), (5, 1, 0, 5, 18)]
NREL2 = len(RELS2)

NSUB = 16
NCORE = 2
CHE = 112
CHUNKS = 28
EPAD = NSUB * CHUNKS * CHE
NACC = 10016
RPT = NACC // NSUB
PADROW = N


def _sc_body(nrel, xt, srci, dsti, zrows, agg, srci_v, dsti_v, g0, g1,
             acc, gsem0, gsem1, ssem0, ssem1):
    c = lax.axis_index("c")
    s = lax.axis_index("s")
    gs = (g0, g1)
    sems = (gsem0, gsem1)
    ssems = (ssem0, ssem1)

    @pl.loop(0, nrel // NCORE)
    def _(i):
        r = i * NCORE + c

        pltpu.sync_copy(zrows.at[pl.ds(s * RPT, RPT)],
                        acc.at[pl.ds(s * RPT, RPT)])
        pltpu.sync_copy(srci.at[r, s], srci_v)
        pltpu.sync_copy(dsti.at[r, s], dsti_v)
        descs = {0: pltpu.async_copy(xt.at[srci_v.at[0]], g0, gsem0)}
        sdescs = {}
        plsc.subcore_barrier()

        for j in range(CHUNKS):
            b = j % 2
            if j + 1 < CHUNKS:
                if j >= 1:
                    sdescs.pop(j - 1).wait()
                descs[j + 1] = pltpu.async_copy(xt.at[srci_v.at[j + 1]],
                                                gs[1 - b], sems[1 - b])
            descs.pop(j).wait()
            sdescs[j] = pltpu.async_copy(gs[b], acc.at[dsti_v.at[j]],
                                         ssems[b], add=True)
        for j in sorted(sdescs):
            sdescs.pop(j).wait()

        plsc.subcore_barrier()
        base = s * RPT
        pltpu.sync_copy(acc.at[pl.ds(base, RPT)], agg.at[r, pl.ds(base, RPT)])


def _make_sc_aggregate(nrel):
    mesh = plsc.VectorSubcoreMesh(core_axis_name="c", subcore_axis_name="s")
    return pl.kernel(
        functools.partial(_sc_body, nrel),
        out_type=jax.ShapeDtypeStruct((nrel, NACC, W), jnp.float32),
        mesh=mesh,
        scratch_types=[
            pltpu.VMEM((CHUNKS, CHE), jnp.int32),
            pltpu.VMEM((CHUNKS, CHE), jnp.int32),
            pltpu.VMEM((CHE, W), jnp.float32),
            pltpu.VMEM((CHE, W), jnp.float32),
            pltpu.VMEM_SHARED((NACC, W), jnp.float32),
            pltpu.SemaphoreType.DMA,
            pltpu.SemaphoreType.DMA,
            pltpu.SemaphoreType.DMA,
            pltpu.SemaphoreType.DMA,
        ],
        compiler_params=pltpu.CompilerParams(use_tc_tiling_on_sc=False),
    )


def _pack_edges(src_rows, dst_rows):
    nrel = src_rows.shape[0]
    src_p = jnp.zeros((nrel, EPAD), jnp.int32).at[:, :E].set(src_rows)
    dst_p = jnp.full((nrel, EPAD), PADROW, jnp.int32).at[:, :E].set(dst_rows)
    return (src_p.reshape(nrel, NSUB, CHUNKS, CHE),
            dst_p.reshape(nrel, NSUB, CHUNKS, CHE))


def _augment(x):
    rows = x.shape[0]
    tail = jnp.zeros((rows, W - D), x.dtype).at[:, 0].set(1.0)
    return jnp.concatenate([x, tail], axis=1)


BR = 2504
RB = NACC // BR


def _d_of_k(k):
    d = jnp.int32(0)
    for f in GROUP_FIRST_K[1:]:
        d = d + (k >= f).astype(jnp.int32)
    return d


def _is_in(k, ks):
    r = k == ks[0]
    for v in ks[1:]:
        r = jnp.logical_or(r, k == v)
    return r


def _hid_tail(n):
    lane = lax.broadcasted_iota(jnp.int32, (n, W - D), 1)
    return jnp.where(lane == 0, 1.0, 0.0).astype(jnp.float32)


def _tc1_body(agg_ref, x_ref, wl_ref, wr_ref, b_ref, out_ref):
    k = pl.program_id(1)
    is_first = _is_in(k, GROUP_FIRST_K)
    is_last = _is_in(k, GROUP_LAST_K)
    a = agg_ref[0]
    inv = 1.0 / jnp.maximum(a[:, D:D + 1], 1.0)
    contrib = jnp.dot((a[:, :D] * inv).astype(jnp.bfloat16), wl_ref[0],
                      preferred_element_type=jnp.float32)

    @pl.when(is_first)
    def _():
        out_ref[0, :, :D] = (jnp.dot(x_ref[0], wr_ref[0],
                                     preferred_element_type=jnp.float32)
                             + b_ref[0] + contrib)

    @pl.when(jnp.logical_not(is_first))
    def _():
        out_ref[0, :, :D] += contrib

    @pl.when(is_last)
    def _():
        out_ref[0, :, :D] = jnp.maximum(out_ref[0, :, :D], 0.0)
        out_ref[0, :, D:] = _hid_tail(BR)


def _tc_layer1(agg, x7, wl, wr, b):
    d_of_k = _d_of_k
    grid = (RB, NREL1)
    return pl.pallas_call(
        _tc1_body,
        grid=grid,
        in_specs=[
            pl.BlockSpec((1, BR, W), lambda rb, k: (k, rb, 0)),
            pl.BlockSpec((1, BR, D), lambda rb, k: (d_of_k(k), rb, 0)),
            pl.BlockSpec((1, D, H), lambda rb, k: (k, 0, 0)),
            pl.BlockSpec((1, D, H), lambda rb, k: (d_of_k(k), 0, 0)),
            pl.BlockSpec((1, 1, H), lambda rb, k: (d_of_k(k), 0, 0)),
        ],
        out_specs=pl.BlockSpec((1, BR, W), lambda rb, k: (d_of_k(k), rb, 0)),
        out_shape=jax.ShapeDtypeStruct((len(DST_TYPES), NACC, W),
                                       jnp.float32),
    )(agg, x7, wl, wr, b)


def _tc2_body(agg_ref, h_ref, wl_ref, wr_ref, b_ref, lw_ref, lb_ref,
              out_ref, acc_ref):
    k = pl.program_id(1)
    a = agg_ref[0]
    inv = 1.0 / jnp.maximum(a[:, D:D + 1], 1.0)
    contrib = jnp.dot((a[:, :D] * inv).astype(jnp.bfloat16), wl_ref[0],
                      preferred_element_type=jnp.float32)

    @pl.when(k == 0)
    def _():
        acc_ref[...] = (jnp.dot(h_ref[0, :, :D].astype(jnp.bfloat16),
                                wr_ref[...],
                                preferred_element_type=jnp.float32)
                        + b_ref[...][None, :] + contrib)

    @pl.when(k > 0)
    def _():
        acc_ref[...] += contrib

    @pl.when(k == NREL2 - 1)
    def _():
        out_ref[...] = (jnp.dot(jnp.maximum(acc_ref[...], 0.0)
                                .astype(jnp.bfloat16), lw_ref[...],
                                preferred_element_type=jnp.float32)
                        + lb_ref[...][None, :])


def _tc_layer2(agg2, h, wl, wr, b, lin_w, lin_b):
    grid = (RB, NREL2)
    return pl.pallas_call(
        _tc2_body,
        grid=grid,
        in_specs=[
            pl.BlockSpec((1, BR, W), lambda rb, k: (k, rb, 0)),
            pl.BlockSpec((1, BR, W), lambda rb, k: (len(DST_TYPES) - 1, rb, 0)),
            pl.BlockSpec((1, H, H), lambda rb, k: (k, 0, 0)),
            pl.BlockSpec((H, H), lambda rb, k: (0, 0)),
            pl.BlockSpec((H,), lambda rb, k: (0,)),
            pl.BlockSpec((H, OUT), lambda rb, k: (0, 0)),
            pl.BlockSpec((OUT,), lambda rb, k: (0,)),
        ],
        out_specs=pl.BlockSpec((BR, OUT), lambda rb, k: (rb, 0)),
        out_shape=jax.ShapeDtypeStruct((NACC, OUT), jnp.float32),
        scratch_shapes=[pltpu.VMEM((BR, H), jnp.float32)],
    )(agg2, h, wl, wr, b, lin_w, lin_b)


def kernel(x_course, x_field, x_resource, x_teacher, x_school, x_user,
           x_comment, x_reply, x_exercise, x_video,
           ei_course_field, ei_course_resource, ei_course_teacher,
           ei_course_school, ei_course_user, ei_course_comment,
           ei_comment_reply, ei_user_comment, ei_user_reply,
           ei_school_user, ei_school_teacher, ei_resource_exercise,
           ei_resource_video, Wl, Wr, bl, lin_W, lin_b):
    xs = {"course": x_course, "field": x_field, "resource": x_resource,
          "teacher": x_teacher, "school": x_school, "user": x_user,
          "comment": x_comment, "reply": x_reply, "exercise": x_exercise,
          "video": x_video}
    eis = [ei_course_field, ei_course_resource, ei_course_teacher,
           ei_course_school, ei_course_user, ei_course_comment,
           ei_comment_reply, ei_user_comment, ei_user_reply,
           ei_school_user, ei_school_teacher, ei_resource_exercise,
           ei_resource_video]
    eis = [e.astype(jnp.int32) for e in eis]

    xt1 = _augment(jnp.concatenate([xs[nt] for nt in NODES], axis=0))
    src1 = jnp.stack([eis[j][sr] + N * NODES.index(snt)
                      for (j, sr, dr, snt, k) in RELS1])
    dst1 = jnp.stack([eis[j][dr] for (j, sr, dr, snt, k) in RELS1])
    srci1, dsti1 = _pack_edges(src1, dst1)
    zrows = jnp.zeros((NACC, W), jnp.float32)
    agg1 = _make_sc_aggregate(NREL1)(xt1, srci1, dsti1, zrows)

    perm1 = [k for (_, _, _, _, k) in RELS1]
    wl1 = Wl[0, jnp.asarray(perm1)].astype(jnp.bfloat16)
    goff = 0
    wr_sums, b_sums = [], []
    for g in GROUP_SIZES:
        ks = jnp.asarray(perm1[goff:goff + g])
        wr_sums.append(Wr[0, ks].sum(axis=0))
        b_sums.append(bl[0, ks].sum(axis=0))
        goff += g
    wr1 = jnp.stack(wr_sums).astype(jnp.bfloat16)
    b1 = jnp.stack(b_sums)[:, None, :]
    pad = ((0, NACC - N), (0, 0))
    x7 = jnp.stack([jnp.pad(xs[nt], pad)
                    for nt in DST_TYPES]).astype(jnp.bfloat16)
    h = _tc_layer1(agg1, x7, wl1, wr1, b1)

    xt2 = h.reshape(len(DST_TYPES) * NACC, W)
    src2 = jnp.stack([eis[j][sr] + NACC * hi
                      for (j, sr, dr, hi, k) in RELS2])
    dst2 = jnp.stack([eis[j][dr] for (j, sr, dr, hi, k) in RELS2])
    srci2, dsti2 = _pack_edges(src2, dst2)
    agg2 = _make_sc_aggregate(NREL2)(xt2, srci2, dsti2, zrows)

    perm2 = jnp.asarray([k for (_, _, _, _, k) in RELS2])
    wl2 = Wl[1, perm2].astype(jnp.bfloat16)
    wr2 = Wr[1, perm2].sum(axis=0).astype(jnp.bfloat16)
    b2 = bl[1, perm2].sum(axis=0)
    y = _tc_layer2(agg2, h, wl2, wr2, b2,
                   lin_W.astype(jnp.bfloat16), lin_b)
    return y[:N]


# speedup vs baseline: 3.6019x; 1.0166x over previous
import functools

import jax
import jax.numpy as jnp
from jax import lax
from jax.experimental import pallas as pl
from jax.experimental.pallas import tpu as pltpu
from jax.experimental.pallas import tpu_sc as plsc

N = 10000
D = 128
H = 128
OUT = 64
E = 50000
W = 144

NODES = ["course", "field", "resource", "teacher", "school", "user",
         "comment", "reply", "exercise", "video"]

DST_TYPES = ["field", "resource", "teacher", "school", "user", "comment",
             "course"]
RELS1 = [
    (0, 0, 1, "course", 0),
    (1, 0, 1, "course", 1), (11, 1, 0, "exercise", 24), (12, 1, 0, "video", 25),
    (2, 0, 1, "course", 2), (10, 0, 1, "school", 10),
    (3, 0, 1, "course", 3), (9, 1, 0, "user", 22), (10, 1, 0, "teacher", 23),
    (4, 0, 1, "course", 4), (9, 0, 1, "school", 9), (7, 1, 0, "comment", 20),
    (8, 1, 0, "reply", 21),
    (5, 0, 1, "course", 5), (7, 0, 1, "user", 7), (6, 1, 0, "reply", 19),
    (0, 1, 0, "field", 13), (1, 1, 0, "resource", 14),
    (2, 1, 0, "teacher", 15), (3, 1, 0, "school", 16),
    (4, 1, 0, "user", 17), (5, 1, 0, "comment", 18),
]
NREL1 = len(RELS1)
GROUP_SIZES = [1, 3, 2, 3, 4, 3, 6]
GROUP_FIRST_K = [0, 1, 4, 6, 9, 13, 16]
GROUP_LAST_K = [0, 3, 5, 8, 12, 15, 21]

RELS2 = [(0, 1, 0, 0, 13), (1, 1, 0, 1, 14), (2, 1, 0, 2, 15),
         (3, 1, 0, 3, 16), (4, 1, 0, 4, 17), (5, 1, 0, 5, 18)]
NREL2 = len(RELS2)

NSUB = 16
NCORE = 2
CHE = 112
CHUNKS = 28
EPAD = NSUB * CHUNKS * CHE
NACC = 10016
RPT = NACC // NSUB
PADROW = N


def _sc_body(nrel, xt, srci, dsti, zrows, agg, srci_v, dsti_v, g0, g1,
             acc, gsem0, gsem1, ssem0, ssem1):
    c = lax.axis_index("c")
    s = lax.axis_index("s")
    gs = (g0, g1)
    sems = (gsem0, gsem1)
    ssems = (ssem0, ssem1)

    @pl.loop(0, nrel // NCORE)
    def _(i):
        r = i * NCORE + c

        pltpu.sync_copy(zrows.at[pl.ds(s * RPT, RPT)],
                        acc.at[pl.ds(s * RPT, RPT)])
        pltpu.sync_copy(srci.at[r, s], srci_v)
        pltpu.sync_copy(dsti.at[r, s], dsti_v)
        descs = {0: pltpu.async_copy(xt.at[srci_v.at[0]], g0, gsem0)}
        sdescs = {}
        plsc.subcore_barrier()

        for j in range(CHUNKS):
            b = j % 2
            if j + 1 < CHUNKS:
                if j >= 1:
                    sdescs.pop(j - 1).wait()
                descs[j + 1] = pltpu.async_copy(xt.at[srci_v.at[j + 1]],
                                                gs[1 - b], sems[1 - b])
            descs.pop(j).wait()
            sdescs[j] = pltpu.async_copy(gs[b], acc.at[dsti_v.at[j]],
                                         ssems[b], add=True)
        for j in sorted(sdescs):
            sdescs.pop(j).wait()

        plsc.subcore_barrier()
        base = s * RPT
        pltpu.sync_copy(acc.at[pl.ds(base, RPT)], agg.at[r, pl.ds(base, RPT)])


def _make_sc_aggregate(nrel):
    mesh = plsc.VectorSubcoreMesh(core_axis_name="c", subcore_axis_name="s")
    return pl.kernel(
        functools.partial(_sc_body, nrel),
        out_type=jax.ShapeDtypeStruct((nrel, NACC, W), jnp.float32),
        mesh=mesh,
        scratch_types=[
            pltpu.VMEM((CHUNKS, CHE), jnp.int32),
            pltpu.VMEM((CHUNKS, CHE), jnp.int32),
            pltpu.VMEM((CHE, W), jnp.float32),
            pltpu.VMEM((CHE, W), jnp.float32),
            pltpu.VMEM_SHARED((NACC, W), jnp.float32),
            pltpu.SemaphoreType.DMA,
            pltpu.SemaphoreType.DMA,
            pltpu.SemaphoreType.DMA,
            pltpu.SemaphoreType.DMA,
        ],
        compiler_params=pltpu.CompilerParams(use_tc_tiling_on_sc=False),
    )


def _pack_edges(src_rows, dst_rows):
    nrel = src_rows.shape[0]
    src_p = jnp.concatenate(
        [src_rows, jnp.zeros((nrel, EPAD - E), jnp.int32)], axis=1)
    dst_p = jnp.concatenate(
        [dst_rows, jnp.full((nrel, EPAD - E), PADROW, jnp.int32)], axis=1)
    return (src_p.reshape(nrel, NSUB, CHUNKS, CHE),
            dst_p.reshape(nrel, NSUB, CHUNKS, CHE))


def _augment(x):
    rows = x.shape[0]
    tail = jnp.zeros((rows, W - D), x.dtype).at[:, 0].set(1.0)
    return jnp.concatenate([x, tail], axis=1)


BR = 2504
RB = NACC // BR


def _d_of_k(k):
    d = jnp.int32(0)
    for f in GROUP_FIRST_K[1:]:
        d = d + (k >= f).astype(jnp.int32)
    return d


def _is_in(k, ks):
    r = k == ks[0]
    for v in ks[1:]:
        r = jnp.logical_or(r, k == v)
    return r


def _hid_tail(n):
    lane = lax.broadcasted_iota(jnp.int32, (n, W - D), 1)
    return jnp.where(lane == 0, 1.0, 0.0).astype(jnp.float32)


def _tc1_body(agg_ref, x_ref, wl_ref, wr_ref, b_ref, out_ref):
    k = pl.program_id(1)
    is_first = _is_in(k, GROUP_FIRST_K)
    is_last = _is_in(k, GROUP_LAST_K)
    a = agg_ref[0]
    inv = 1.0 / jnp.maximum(a[:, D:D + 1], 1.0)
    contrib = jnp.dot((a[:, :D] * inv).astype(jnp.bfloat16), wl_ref[0],
                      preferred_element_type=jnp.float32)

    @pl.when(is_first)
    def _():
        out_ref[0, :, :D] = (jnp.dot(x_ref[0], wr_ref[0],
                                     preferred_element_type=jnp.float32)
                             + b_ref[0] + contrib)

    @pl.when(jnp.logical_not(is_first))
    def _():
        out_ref[0, :, :D] += contrib

    @pl.when(is_last)
    def _():
        out_ref[0, :, :D] = jnp.maximum(out_ref[0, :, :D], 0.0)
        out_ref[0, :, D:] = _hid_tail(BR)


def _tc_layer1(agg, x7, wl, wr, b):
    d_of_k = _d_of_k
    grid = (RB, NREL1)
    return pl.pallas_call(
        _tc1_body,
        grid=grid,
        in_specs=[
            pl.BlockSpec((1, BR, W), lambda rb, k: (k, rb, 0)),
            pl.BlockSpec((1, BR, D), lambda rb, k: (d_of_k(k), rb, 0)),
            pl.BlockSpec((1, D, H), lambda rb, k: (k, 0, 0)),
            pl.BlockSpec((1, D, H), lambda rb, k: (d_of_k(k), 0, 0)),
            pl.BlockSpec((1, 1, H), lambda rb, k: (d_of_k(k), 0, 0)),
        ],
        out_specs=pl.BlockSpec((1, BR, W), lambda rb, k: (d_of_k(k), rb, 0)),
        out_shape=jax.ShapeDtypeStruct((len(DST_TYPES), NACC, W),
                                       jnp.float32),
    )(agg, x7, wl, wr, b)


def _tc2_body(agg_ref, h_ref, wl_ref, wr_ref, b_ref, lw_ref, lb_ref,
              out_ref, acc_ref):
    k = pl.program_id(1)
    a = agg_ref[0]
    inv = 1.0 / jnp.maximum(a[:, D:D + 1], 1.0)
    contrib = jnp.dot((a[:, :D] * inv).astype(jnp.bfloat16), wl_ref[0],
                      preferred_element_type=jnp.float32)

    @pl.when(k == 0)
    def _():
        acc_ref[...] = (jnp.dot(h_ref[0, :, :D].astype(jnp.bfloat16),
                                wr_ref[...],
                                preferred_element_type=jnp.float32)
                        + b_ref[...][None, :] + contrib)

    @pl.when(k > 0)
    def _():
        acc_ref[...] += contrib

    @pl.when(k == NREL2 - 1)
    def _():
        out_ref[...] = (jnp.dot(jnp.maximum(acc_ref[...], 0.0)
                                .astype(jnp.bfloat16), lw_ref[...],
                                preferred_element_type=jnp.float32)
                        + lb_ref[...][None, :])


def _tc_layer2(agg2, h, wl, wr, b, lin_w, lin_b):
    grid = (RB, NREL2)
    return pl.pallas_call(
        _tc2_body,
        grid=grid,
        in_specs=[
            pl.BlockSpec((1, BR, W), lambda rb, k: (k, rb, 0)),
            pl.BlockSpec((1, BR, W), lambda rb, k: (len(DST_TYPES) - 1, rb, 0)),
            pl.BlockSpec((1, H, H), lambda rb, k: (k, 0, 0)),
            pl.BlockSpec((H, H), lambda rb, k: (0, 0)),
            pl.BlockSpec((H,), lambda rb, k: (0,)),
            pl.BlockSpec((H, OUT), lambda rb, k: (0, 0)),
            pl.BlockSpec((OUT,), lambda rb, k: (0,)),
        ],
        out_specs=pl.BlockSpec((BR, OUT), lambda rb, k: (rb, 0)),
        out_shape=jax.ShapeDtypeStruct((NACC, OUT), jnp.float32),
        scratch_shapes=[pltpu.VMEM((BR, H), jnp.float32)],
    )(agg2, h, wl, wr, b, lin_w, lin_b)


def kernel(x_course, x_field, x_resource, x_teacher, x_school, x_user,
           x_comment, x_reply, x_exercise, x_video,
           ei_course_field, ei_course_resource, ei_course_teacher,
           ei_course_school, ei_course_user, ei_course_comment,
           ei_comment_reply, ei_user_comment, ei_user_reply,
           ei_school_user, ei_school_teacher, ei_resource_exercise,
           ei_resource_video, Wl, Wr, bl, lin_W, lin_b):
    xs = {"course": x_course, "field": x_field, "resource": x_resource,
          "teacher": x_teacher, "school": x_school, "user": x_user,
          "comment": x_comment, "reply": x_reply, "exercise": x_exercise,
          "video": x_video}
    eis = [ei_course_field, ei_course_resource, ei_course_teacher,
           ei_course_school, ei_course_user, ei_course_comment,
           ei_comment_reply, ei_user_comment, ei_user_reply,
           ei_school_user, ei_school_teacher, ei_resource_exercise,
           ei_resource_video]
    eis = [e.astype(jnp.int32) for e in eis]

    xt1 = _augment(jnp.concatenate([xs[nt] for nt in NODES], axis=0))
    src1 = jnp.stack([eis[j][sr] + N * NODES.index(snt)
                      for (j, sr, dr, snt, k) in RELS1])
    dst1 = jnp.stack([eis[j][dr] for (j, sr, dr, snt, k) in RELS1])
    srci1, dsti1 = _pack_edges(src1, dst1)
    zrows = jnp.zeros((NACC, W), jnp.float32)
    agg1 = _make_sc_aggregate(NREL1)(xt1, srci1, dsti1, zrows)

    perm1 = [k for (_, _, _, _, k) in RELS1]
    wl1 = jnp.stack([Wl[0, k] for k in perm1]).astype(jnp.bfloat16)
    goff = 0
    wr_sums, b_sums = [], []
    for g in GROUP_SIZES:
        ks = perm1[goff:goff + g]
        wr_sums.append(sum(Wr[0, k] for k in ks))
        b_sums.append(sum(bl[0, k] for k in ks))
        goff += g
    wr1 = jnp.stack(wr_sums).astype(jnp.bfloat16)
    b1 = jnp.stack(b_sums)[:, None, :]
    pad = ((0, NACC - N), (0, 0))
    x7 = jnp.stack([jnp.pad(xs[nt], pad)
                    for nt in DST_TYPES]).astype(jnp.bfloat16)
    h = _tc_layer1(agg1, x7, wl1, wr1, b1)

    xt2 = h.reshape(len(DST_TYPES) * NACC, W)
    src2 = jnp.stack([eis[j][sr] + NACC * hi
                      for (j, sr, dr, hi, k) in RELS2])
    dst2 = jnp.stack([eis[j][dr] for (j, sr, dr, hi, k) in RELS2])
    srci2, dsti2 = _pack_edges(src2, dst2)
    agg2 = _make_sc_aggregate(NREL2)(xt2, srci2, dsti2, zrows)

    perm2 = [k for (_, _, _, _, k) in RELS2]
    wl2 = jnp.stack([Wl[1, k] for k in perm2]).astype(jnp.bfloat16)
    wr2 = sum(Wr[1, k] for k in perm2).astype(jnp.bfloat16)
    b2 = sum(bl[1, k] for k in perm2)
    y = _tc_layer2(agg2, h, wl2, wr2, b2,
                   lin_W.astype(jnp.bfloat16), lin_b)
    return y[:N]


# speedup vs baseline: 4.0762x; 1.1317x over previous
import functools

import jax
import jax.numpy as jnp
from jax import lax
from jax.experimental import pallas as pl
from jax.experimental.pallas import tpu as pltpu
from jax.experimental.pallas import tpu_sc as plsc

N = 10000
D = 128
H = 128
OUT = 64
E = 50000
W = 144

NODES = ["course", "field", "resource", "teacher", "school", "user",
         "comment", "reply", "exercise", "video"]

DST_TYPES = ["field", "resource", "teacher", "school", "user", "comment",
             "course"]
RELS1 = [
    (0, 0, 1, "course", 0),
    (1, 0, 1, "course", 1), (11, 1, 0, "exercise", 24), (12, 1, 0, "video", 25),
    (2, 0, 1, "course", 2), (10, 0, 1, "school", 10),
    (3, 0, 1, "course", 3), (9, 1, 0, "user", 22), (10, 1, 0, "teacher", 23),
    (4, 0, 1, "course", 4), (9, 0, 1, "school", 9), (7, 1, 0, "comment", 20),
    (8, 1, 0, "reply", 21),
    (5, 0, 1, "course", 5), (7, 0, 1, "user", 7), (6, 1, 0, "reply", 19),
    (0, 1, 0, "field", 13), (1, 1, 0, "resource", 14),
    (2, 1, 0, "teacher", 15), (3, 1, 0, "school", 16),
    (4, 1, 0, "user", 17), (5, 1, 0, "comment", 18),
]
NREL1 = len(RELS1)
NREL1A = 16
NREL1B = 6
GROUP_SIZES = [1, 3, 2, 3, 4, 3, 6]
GROUP_FIRST_K = [0, 1, 4, 6, 9, 13, 16]
GROUP_LAST_K = [0, 3, 5, 8, 12, 15, 21]

RELS2 = [(0, 1, 0, 0, 13), (1, 1, 0, 1, 14), (2, 1, 0, 2, 15),
         (3, 1, 0, 3, 16), (4, 1, 0, 4, 17), (5, 1, 0, 5, 18)]
NREL2 = len(RELS2)

NSUB = 16
NCORE = 2
CHE = 112
CHUNKS = 28
EPAD = NSUB * CHUNKS * CHE
NACC = 10016
RPT = NACC // NSUB
PADROW = N


def _sc_body(nrel, xt, srci, dsti, zrows, agg, srci_v, dsti_v, g0, g1,
             acc, gsem0, gsem1, ssem0, ssem1):
    c = lax.axis_index("c")
    s = lax.axis_index("s")
    gs = (g0, g1)
    sems = (gsem0, gsem1)
    ssems = (ssem0, ssem1)

    @pl.loop(0, nrel // NCORE)
    def _(i):
        r = i * NCORE + c

        pltpu.sync_copy(zrows.at[pl.ds(s * RPT, RPT)],
                        acc.at[pl.ds(s * RPT, RPT)])
        pltpu.sync_copy(srci.at[r, s], srci_v)
        pltpu.sync_copy(dsti.at[r, s], dsti_v)
        descs = {0: pltpu.async_copy(xt.at[srci_v.at[0]], g0, gsem0)}
        sdescs = {}
        plsc.subcore_barrier()

        for j in range(CHUNKS):
            b = j % 2
            if j + 1 < CHUNKS:
                if j >= 1:
                    sdescs.pop(j - 1).wait()
                descs[j + 1] = pltpu.async_copy(xt.at[srci_v.at[j + 1]],
                                                gs[1 - b], sems[1 - b])
            descs.pop(j).wait()
            sdescs[j] = pltpu.async_copy(gs[b], acc.at[dsti_v.at[j]],
                                         ssems[b], add=True)
        for j in sorted(sdescs):
            sdescs.pop(j).wait()

        plsc.subcore_barrier()
        base = s * RPT
        pltpu.sync_copy(acc.at[pl.ds(base, RPT)], agg.at[r, pl.ds(base, RPT)])


def _make_sc_aggregate(nrel):
    mesh = plsc.VectorSubcoreMesh(core_axis_name="c", subcore_axis_name="s")
    return pl.kernel(
        functools.partial(_sc_body, nrel),
        out_type=jax.ShapeDtypeStruct((nrel, NACC, W), jnp.float32),
        mesh=mesh,
        scratch_types=[
            pltpu.VMEM((CHUNKS, CHE), jnp.int32),
            pltpu.VMEM((CHUNKS, CHE), jnp.int32),
            pltpu.VMEM((CHE, W), jnp.float32),
            pltpu.VMEM((CHE, W), jnp.float32),
            pltpu.VMEM_SHARED((NACC, W), jnp.float32),
            pltpu.SemaphoreType.DMA,
            pltpu.SemaphoreType.DMA,
            pltpu.SemaphoreType.DMA,
            pltpu.SemaphoreType.DMA,
        ],
        compiler_params=pltpu.CompilerParams(use_tc_tiling_on_sc=False),
    )


def _pack_edges(src_rows, dst_rows):
    nrel = src_rows.shape[0]
    src_p = jnp.concatenate(
        [src_rows, jnp.zeros((nrel, EPAD - E), jnp.int32)], axis=1)
    dst_p = jnp.concatenate(
        [dst_rows, jnp.full((nrel, EPAD - E), PADROW, jnp.int32)], axis=1)
    return (src_p.reshape(nrel, NSUB, CHUNKS, CHE),
            dst_p.reshape(nrel, NSUB, CHUNKS, CHE))


def _augment(x):
    rows = x.shape[0]
    tail = jnp.zeros((rows, W - D), x.dtype).at[:, 0].set(1.0)
    return jnp.concatenate([x, tail], axis=1)


BR = 2504
RB = NACC // BR


def _d_of_k(k, firsts):
    d = jnp.int32(0)
    for f in firsts:
        d = d + (k >= f).astype(jnp.int32)
    return d


def _is_in(k, ks):
    r = k == ks[0]
    for v in ks[1:]:
        r = jnp.logical_or(r, k == v)
    return r


def _hid_tail(n):
    lane = lax.broadcasted_iota(jnp.int32, (n, W - D), 1)
    return jnp.where(lane == 0, 1.0, 0.0).astype(jnp.float32)


def _tc1a_body(agg_ref, x_ref, wl_ref, wr_ref, b_ref, out_ref):
    k = pl.program_id(1)
    is_first = _is_in(k, GROUP_FIRST_K[:-1])
    is_last = _is_in(k, GROUP_LAST_K[:-1])
    a = agg_ref[0]
    inv = 1.0 / jnp.maximum(a[:, D:D + 1], 1.0)
    contrib = jnp.dot((a[:, :D] * inv).astype(jnp.bfloat16), wl_ref[0],
                      preferred_element_type=jnp.float32)

    @pl.when(is_first)
    def _():
        out_ref[0, :, :D] = (jnp.dot(x_ref[0], wr_ref[0],
                                     preferred_element_type=jnp.float32)
                             + b_ref[0] + contrib)

    @pl.when(jnp.logical_not(is_first))
    def _():
        out_ref[0, :, :D] += contrib

    @pl.when(is_last)
    def _():
        out_ref[0, :, :D] = jnp.maximum(out_ref[0, :, :D], 0.0)
        out_ref[0, :, D:] = _hid_tail(BR)


def _tc_layer1a(agg, x6, wl, wr, b):
    d_of_k = functools.partial(_d_of_k, firsts=GROUP_FIRST_K[1:-1])
    grid = (RB, NREL1A)
    return pl.pallas_call(
        _tc1a_body,
        grid=grid,
        in_specs=[
            pl.BlockSpec((1, BR, W), lambda rb, k: (k, rb, 0)),
            pl.BlockSpec((1, BR, D), lambda rb, k: (d_of_k(k), rb, 0)),
            pl.BlockSpec((1, D, H), lambda rb, k: (k, 0, 0)),
            pl.BlockSpec((1, D, H), lambda rb, k: (d_of_k(k), 0, 0)),
            pl.BlockSpec((1, 1, H), lambda rb, k: (d_of_k(k), 0, 0)),
        ],
        out_specs=pl.BlockSpec((1, BR, W), lambda rb, k: (d_of_k(k), rb, 0)),
        out_shape=jax.ShapeDtypeStruct((len(DST_TYPES) - 1, NACC, W),
                                       jnp.float32),
    )(agg, x6, wl, wr, b)


def _tc1b_body(agg_ref, x_ref, wl_ref, wr_ref, b_ref, out_ref):
    k = pl.program_id(1)
    a = agg_ref[0]
    inv = 1.0 / jnp.maximum(a[:, D:D + 1], 1.0)
    contrib = jnp.dot((a[:, :D] * inv).astype(jnp.bfloat16), wl_ref[0],
                      preferred_element_type=jnp.float32)

    @pl.when(k == 0)
    def _():
        out_ref[...] = (jnp.dot(x_ref[...], wr_ref[...],
                                preferred_element_type=jnp.float32)
                        + b_ref[...][None, :] + contrib)

    @pl.when(k > 0)
    def _():
        out_ref[...] += contrib

    @pl.when(k == NREL1B - 1)
    def _():
        out_ref[...] = jnp.maximum(out_ref[...], 0.0)


def _tc_layer1b(agg, xc, wl, wr, b):
    grid = (RB, NREL1B)
    return pl.pallas_call(
        _tc1b_body,
        grid=grid,
        in_specs=[
            pl.BlockSpec((1, BR, W), lambda rb, k: (k, rb, 0)),
            pl.BlockSpec((BR, D), lambda rb, k: (rb, 0)),
            pl.BlockSpec((1, D, H), lambda rb, k: (k, 0, 0)),
            pl.BlockSpec((D, H), lambda rb, k: (0, 0)),
            pl.BlockSpec((H,), lambda rb, k: (0,)),
        ],
        out_specs=pl.BlockSpec((BR, H), lambda rb, k: (rb, 0)),
        out_shape=jax.ShapeDtypeStruct((NACC, H), jnp.float32),
    )(agg, xc, wl, wr, b)


def _tc2_body(agg_ref, h_ref, wl_ref, wr_ref, b_ref, lw_ref, lb_ref,
              out_ref, acc_ref):
    k = pl.program_id(1)
    a = agg_ref[0]
    inv = 1.0 / jnp.maximum(a[:, D:D + 1], 1.0)
    contrib = jnp.dot((a[:, :D] * inv).astype(jnp.bfloat16), wl_ref[0],
                      preferred_element_type=jnp.float32)

    @pl.when(k == 0)
    def _():
        acc_ref[...] = (jnp.dot(h_ref[...].astype(jnp.bfloat16),
                                wr_ref[...],
                                preferred_element_type=jnp.float32)
                        + b_ref[...][None, :] + contrib)

    @pl.when(k > 0)
    def _():
        acc_ref[...] += contrib

    @pl.when(k == NREL2 - 1)
    def _():
        out_ref[...] = (jnp.dot(jnp.maximum(acc_ref[...], 0.0)
                                .astype(jnp.bfloat16), lw_ref[...],
                                preferred_element_type=jnp.float32)
                        + lb_ref[...][None, :])


def _tc_layer2(agg2, hc, wl, wr, b, lin_w, lin_b):
    grid = (RB, NREL2)
    return pl.pallas_call(
        _tc2_body,
        grid=grid,
        in_specs=[
            pl.BlockSpec((1, BR, W), lambda rb, k: (k, rb, 0)),
            pl.BlockSpec((BR, D), lambda rb, k: (rb, 0)),
            pl.BlockSpec((1, H, H), lambda rb, k: (k, 0, 0)),
            pl.BlockSpec((H, H), lambda rb, k: (0, 0)),
            pl.BlockSpec((H,), lambda rb, k: (0,)),
            pl.BlockSpec((H, OUT), lambda rb, k: (0, 0)),
            pl.BlockSpec((OUT,), lambda rb, k: (0,)),
        ],
        out_specs=pl.BlockSpec((BR, OUT), lambda rb, k: (rb, 0)),
        out_shape=jax.ShapeDtypeStruct((NACC, OUT), jnp.float32),
        scratch_shapes=[pltpu.VMEM((BR, H), jnp.float32)],
    )(agg2, hc, wl, wr, b, lin_w, lin_b)


def kernel(x_course, x_field, x_resource, x_teacher, x_school, x_user,
           x_comment, x_reply, x_exercise, x_video,
           ei_course_field, ei_course_resource, ei_course_teacher,
           ei_course_school, ei_course_user, ei_course_comment,
           ei_comment_reply, ei_user_comment, ei_user_reply,
           ei_school_user, ei_school_teacher, ei_resource_exercise,
           ei_resource_video, Wl, Wr, bl, lin_W, lin_b):
    xs = {"course": x_course, "field": x_field, "resource": x_resource,
          "teacher": x_teacher, "school": x_school, "user": x_user,
          "comment": x_comment, "reply": x_reply, "exercise": x_exercise,
          "video": x_video}
    eis = [ei_course_field, ei_course_resource, ei_course_teacher,
           ei_course_school, ei_course_user, ei_course_comment,
           ei_comment_reply, ei_user_comment, ei_user_reply,
           ei_school_user, ei_school_teacher, ei_resource_exercise,
           ei_resource_video]
    eis = [e.astype(jnp.int32) for e in eis]

    xt1 = _augment(jnp.concatenate([xs[nt] for nt in NODES], axis=0))
    src1 = jnp.stack([eis[j][sr] + N * NODES.index(snt)
                      for (j, sr, dr, snt, k) in RELS1])
    dst1 = jnp.stack([eis[j][dr] for (j, sr, dr, snt, k) in RELS1])
    srciA, dstiA = _pack_edges(src1[:NREL1A], dst1[:NREL1A])
    srciB, dstiB = _pack_edges(src1[NREL1A:], dst1[NREL1A:])
    zrows = jnp.zeros((NACC, W), jnp.float32)
    aggA = _make_sc_aggregate(NREL1A)(xt1, srciA, dstiA, zrows)
    aggB = _make_sc_aggregate(NREL1B)(xt1, srciB, dstiB, zrows)

    perm1 = [k for (_, _, _, _, k) in RELS1]
    goff = 0
    wr_sums, b_sums = [], []
    for g in GROUP_SIZES:
        ks = perm1[goff:goff + g]
        wr_sums.append(sum(Wr[0, k] for k in ks))
        b_sums.append(sum(bl[0, k] for k in ks))
        goff += g
    wl1a = jnp.stack([Wl[0, k] for k in perm1[:NREL1A]]).astype(jnp.bfloat16)
    wr6 = jnp.stack(wr_sums[:-1]).astype(jnp.bfloat16)
    b6 = jnp.stack(b_sums[:-1])[:, None, :]
    pad = ((0, NACC - N), (0, 0))
    x6 = jnp.stack([jnp.pad(xs[nt], pad)
                    for nt in DST_TYPES[:-1]]).astype(jnp.bfloat16)
    h6 = _tc_layer1a(aggA, x6, wl1a, wr6, b6)

    wl1b = jnp.stack([Wl[0, k] for k in perm1[NREL1A:]]).astype(jnp.bfloat16)
    xc = jnp.pad(xs["course"], pad).astype(jnp.bfloat16)
    hc = _tc_layer1b(aggB, xc, wl1b,
                     wr_sums[-1].astype(jnp.bfloat16), b_sums[-1])

    xt2 = h6.reshape(NREL1B * NACC, W)
    src2 = jnp.stack([eis[j][sr] + NACC * hi
                      for (j, sr, dr, hi, k) in RELS2])
    dst2 = jnp.stack([eis[j][dr] for (j, sr, dr, hi, k) in RELS2])
    srci2, dsti2 = _pack_edges(src2, dst2)
    agg2 = _make_sc_aggregate(NREL2)(xt2, srci2, dsti2, zrows)

    perm2 = [k for (_, _, _, _, k) in RELS2]
    wl2 = jnp.stack([Wl[1, k] for k in perm2]).astype(jnp.bfloat16)
    wr2 = sum(Wr[1, k] for k in perm2).astype(jnp.bfloat16)
    b2 = sum(bl[1, k] for k in perm2)
    y = _tc_layer2(agg2, hc, wl2, wr2, b2,
                   lin_W.astype(jnp.bfloat16), lin_b)
    return y[:N]
